# baseline (speedup 1.0000x reference)
import functools

import jax
import jax.numpy as jnp
from jax import lax
from jax.experimental import pallas as pl
from jax.experimental.pallas import tpu as pltpu
from jax.experimental.pallas import tpu_sc as plsc

_N = 10000
_D = 128
_E = 320000
_NPAD = 10112
_NCORES = 2
_NSUB = 16
_NTILES = _NCORES * _NSUB
_B = 64
_NBUF = 3
_NCHUNK = 54
_NB = _NBUF * _NCHUNK
_EPT = _NB * _B
_RPT = _NPAD // _NSUB


@functools.partial(
    pl.kernel,
    mesh=plsc.VectorSubcoreMesh(core_axis_name="c", subcore_axis_name="s"),
    out_type=jax.ShapeDtypeStruct((_NCORES * _NPAD, _D), jnp.float32),
    scratch_types=[
        pltpu.VMEM((_NB, _B), jnp.int32),
        pltpu.VMEM((_NBUF, _B), jnp.int32),
        pltpu.VMEM((_NBUF, _B), jnp.int32),
        pltpu.VMEM((_NBUF, _B, _D), jnp.float32),
        pltpu.SemaphoreType.DMA((_NBUF,)),
        pltpu.VMEM_SHARED((_NPAD, _D), jnp.float32),
    ],
)
def _edge_pass(u_hbm, pk_hbm, z_hbm, agg_hbm, pk_v, cst, rst, bufs, gsem, acc):
    c = lax.axis_index("c")
    s = lax.axis_index("s")
    w = c * _NSUB + s
    base = s * _RPT

    pltpu.sync_copy(z_hbm.at[pl.ds(base, _RPT)], acc.at[pl.ds(base, _RPT)])
    pltpu.sync_copy(pk_hbm.at[w], pk_v)

    plsc.subcore_barrier()

    def _unpack(g, b):
        for k in range(_B // 16):
            pk = pk_v[g, pl.ds(k * 16, 16)]
            cst[b, pl.ds(k * 16, 16)] = pk & 0xFFFF
            rst[b, pl.ds(k * 16, 16)] = lax.shift_right_logical(pk, 16)

    for b in range(_NBUF):
        _unpack(b, b)
        pltpu.async_copy(u_hbm.at[cst.at[b]], bufs.at[b], gsem.at[b])

    def _chunk(jc, carry):
        for b in range(_NBUF):
            g = jc * _NBUF + b
            pltpu.make_async_copy(
                u_hbm.at[cst.at[b]], bufs.at[b], gsem.at[b]).wait()
            pltpu.sync_copy(bufs.at[b], acc.at[rst.at[b]], add=True)

            @pl.when(jc < _NCHUNK - 1)
            def _():
                gn = jnp.minimum(g + _NBUF, _NB - 1)
                _unpack(gn, b)
                pltpu.async_copy(u_hbm.at[cst.at[b]], bufs.at[b], gsem.at[b])
        return carry

    lax.fori_loop(0, _NCHUNK, _chunk, 0)

    plsc.subcore_barrier()

    pltpu.sync_copy(acc.at[pl.ds(base, _RPT)],
                    agg_hbm.at[pl.ds(c * _NPAD + base, _RPT)])


_R = 632
_GRID = _NPAD // _R


def _prep_body(feat_b, agg0_b, agg1_b, u0_b, a_b, c_b, sqd_b):
    deg = agg0_b[...] + agg1_b[...] + 1.0
    dinv = lax.rsqrt(deg)
    u0_b[...] = feat_b[...] * dinv
    inv = 1.0 / deg
    a_b[...] = 0.5 + 0.5 * inv
    c_b[...] = 0.5 * inv
    sqd_b[...] = deg * dinv


def _combine_body(u_b, agg0_b, agg1_b, a_b, c_b, out_b):
    out_b[...] = a_b[...] * u_b[...] + c_b[...] * (agg0_b[...] + agg1_b[...])


def _final_body(u_b, sqd_b, wt_b, bias_b, out_b):
    h = u_b[...] * sqd_b[...]
    out_b[...] = jnp.dot(h, wt_b[...],
                         preferred_element_type=jnp.float32) + bias_b[...]


def _row_spec():
    return pl.BlockSpec((_R, _D), lambda i: (i, 0))


def _agg_specs():
    return [pl.BlockSpec((_R, _D), lambda i: (i, 0)),
            pl.BlockSpec((_R, _D), lambda i: (i + _GRID, 0))]


_prep_call = pl.pallas_call(
    _prep_body,
    grid=(_GRID,),
    in_specs=[_row_spec()] + _agg_specs(),
    out_specs=[_row_spec()] * 4,
    out_shape=[jax.ShapeDtypeStruct((_NPAD, _D), jnp.float32)] * 4,
)

_combine_call = pl.pallas_call(
    _combine_body,
    grid=(_GRID,),
    in_specs=[_row_spec()] + _agg_specs() + [_row_spec(), _row_spec()],
    out_specs=_row_spec(),
    out_shape=jax.ShapeDtypeStruct((_NPAD, _D), jnp.float32),
)

_final_call = pl.pallas_call(
    _final_body,
    grid=(_GRID,),
    in_specs=[_row_spec(), _row_spec(),
              pl.BlockSpec((_D, _D), lambda i: (0, 0)),
              pl.BlockSpec((1, _D), lambda i: (0, 0))],
    out_specs=_row_spec(),
    out_shape=jax.ShapeDtypeStruct((_NPAD, _D), jnp.float32),
)


def kernel(feat, edge_index, W, b):
    row = edge_index[0]
    col = edge_index[1]
    pad = _NTILES * _EPT - _E
    sink = jnp.full((pad,), _NPAD - 1, jnp.int32)
    packed = jnp.concatenate(
        [(row << 16) | col, (sink << 16) | sink]).reshape(_NTILES, _NB, _B)

    featp = jnp.pad(feat, ((0, _NPAD - _N), (0, 0)))
    ones = jnp.pad(jnp.ones((_N, _D), jnp.float32), ((0, _NPAD - _N), (0, 0)))
    zeros = jnp.zeros((_NPAD, _D), jnp.float32)

    agg_deg = _edge_pass(ones, packed, zeros)
    u, a, cf, sqd = _prep_call(featp, agg_deg, agg_deg)

    for _ in range(8):
        agg = _edge_pass(u, packed, zeros)
        u = _combine_call(u, agg, agg, a, cf)

    out = _final_call(u, sqd, W.T, b.reshape(1, _D))
    return out[:_N]


# speedup vs baseline: 1.0006x; 1.0006x over previous
import functools

import jax
import jax.numpy as jnp
from jax import lax
from jax.experimental import pallas as pl
from jax.experimental.pallas import tpu as pltpu
from jax.experimental.pallas import tpu_sc as plsc

_N = 10000
_D = 128
_E = 320000
_NPAD = 10112
_NCORES = 2
_NSUB = 16
_NTILES = _NCORES * _NSUB
_B = 64
_NBUF = 3
_NCHUNK = 54
_NB = _NBUF * _NCHUNK
_EPT = _NB * _B
_RPT = _NPAD // _NSUB


@functools.partial(
    pl.kernel,
    mesh=plsc.VectorSubcoreMesh(core_axis_name="c", subcore_axis_name="s"),
    out_type=jax.ShapeDtypeStruct((_NCORES * _NPAD, _D), jnp.float32),
    scratch_types=[
        pltpu.VMEM((_NB, _B), jnp.int32),
        pltpu.VMEM((_NBUF, _B), jnp.int32),
        pltpu.VMEM((_NBUF, _B), jnp.int32),
        pltpu.VMEM((_NBUF, _B, _D), jnp.float32),
        pltpu.SemaphoreType.DMA((_NBUF,)),
        pltpu.SemaphoreType.DMA((_NBUF,)),
        pltpu.VMEM_SHARED((_NPAD, _D), jnp.float32),
    ],
)
def _edge_pass(u_hbm, pk_hbm, z_hbm, agg_hbm,
               pk_v, cst, rst, bufs, gsem, ssem, acc):
    c = lax.axis_index("c")
    s = lax.axis_index("s")
    w = c * _NSUB + s
    base = s * _RPT

    pltpu.sync_copy(z_hbm.at[pl.ds(base, _RPT)], acc.at[pl.ds(base, _RPT)])
    pltpu.sync_copy(pk_hbm.at[w], pk_v)

    plsc.subcore_barrier()

    def _unpack(g, b):
        for k in range(_B // 16):
            pk = pk_v[g, pl.ds(k * 16, 16)]
            cst[b, pl.ds(k * 16, 16)] = pk & 0xFFFF
            rst[b, pl.ds(k * 16, 16)] = lax.shift_right_logical(pk, 16)

    def _fire_gather(g, b):
        _unpack(g, b)
        pltpu.async_copy(u_hbm.at[cst.at[b]], bufs.at[b], gsem.at[b])

    def _wait_gather(b):
        pltpu.make_async_copy(
            u_hbm.at[cst.at[b]], bufs.at[b], gsem.at[b]).wait()

    def _fire_scatter(b):
        pltpu.async_copy(bufs.at[b], acc.at[rst.at[b]], ssem.at[b], add=True)

    def _wait_scatter(b):
        pltpu.make_async_copy(
            bufs.at[b], acc.at[rst.at[b]], ssem.at[b]).wait()

    _fire_gather(0, 0)
    _fire_gather(1, 1)
    _fire_gather(2, 2)
    _wait_gather(0)
    _fire_scatter(0)
    _wait_scatter(0)
    _fire_gather(3, 0)
    _wait_gather(1)
    _fire_scatter(1)
    _wait_scatter(1)
    _fire_gather(4, 1)
    _wait_gather(2)
    _fire_scatter(2)

    def _steps(jc, carry):
        for b in range(_NBUF):
            g = jc * _NBUF + b
            bg = (b + 2) % _NBUF
            bs = b

            @pl.when(g < _NB - 2)
            def _():
                _wait_scatter(bg)
                _fire_gather(g + 2, bg)
            _wait_gather(bs)
            _fire_scatter(bs)
        return carry

    lax.fori_loop(1, _NCHUNK, _steps, 0)

    for b in range(_NBUF):
        _wait_scatter(b)

    plsc.subcore_barrier()

    pltpu.sync_copy(acc.at[pl.ds(base, _RPT)],
                    agg_hbm.at[pl.ds(c * _NPAD + base, _RPT)])


_R = 632
_GRID = _NPAD // _R


def _prep_body(feat_b, agg0_b, agg1_b, u0_b, a_b, c_b, sqd_b):
    deg = agg0_b[...] + agg1_b[...] + 1.0
    dinv = lax.rsqrt(deg)
    u0_b[...] = feat_b[...] * dinv
    inv = 1.0 / deg
    a_b[...] = 0.5 + 0.5 * inv
    c_b[...] = 0.5 * inv
    sqd_b[...] = deg * dinv


def _combine_body(u_b, agg0_b, agg1_b, a_b, c_b, out_b):
    out_b[...] = a_b[...] * u_b[...] + c_b[...] * (agg0_b[...] + agg1_b[...])


def _final_body(u_b, sqd_b, wt_b, bias_b, out_b):
    h = u_b[...] * sqd_b[...]
    out_b[...] = jnp.dot(h, wt_b[...],
                         preferred_element_type=jnp.float32) + bias_b[...]


def _row_spec():
    return pl.BlockSpec((_R, _D), lambda i: (i, 0))


def _agg_specs():
    return [pl.BlockSpec((_R, _D), lambda i: (i, 0)),
            pl.BlockSpec((_R, _D), lambda i: (i + _GRID, 0))]


_prep_call = pl.pallas_call(
    _prep_body,
    grid=(_GRID,),
    in_specs=[_row_spec()] + _agg_specs(),
    out_specs=[_row_spec()] * 4,
    out_shape=[jax.ShapeDtypeStruct((_NPAD, _D), jnp.float32)] * 4,
)

_combine_call = pl.pallas_call(
    _combine_body,
    grid=(_GRID,),
    in_specs=[_row_spec()] + _agg_specs() + [_row_spec(), _row_spec()],
    out_specs=_row_spec(),
    out_shape=jax.ShapeDtypeStruct((_NPAD, _D), jnp.float32),
)

_final_call = pl.pallas_call(
    _final_body,
    grid=(_GRID,),
    in_specs=[_row_spec(), _row_spec(),
              pl.BlockSpec((_D, _D), lambda i: (0, 0)),
              pl.BlockSpec((1, _D), lambda i: (0, 0))],
    out_specs=_row_spec(),
    out_shape=jax.ShapeDtypeStruct((_NPAD, _D), jnp.float32),
)


def kernel(feat, edge_index, W, b):
    row = edge_index[0]
    col = edge_index[1]
    pad = _NTILES * _EPT - _E
    sink = jnp.full((pad,), _NPAD - 1, jnp.int32)
    packed = jnp.concatenate(
        [(row << 16) | col, (sink << 16) | sink]).reshape(_NTILES, _NB, _B)

    featp = jnp.pad(feat, ((0, _NPAD - _N), (0, 0)))
    ones = jnp.pad(jnp.ones((_N, _D), jnp.float32), ((0, _NPAD - _N), (0, 0)))
    zeros = jnp.zeros((_NPAD, _D), jnp.float32)

    agg_deg = _edge_pass(ones, packed, zeros)
    u, a, cf, sqd = _prep_call(featp, agg_deg, agg_deg)

    for _ in range(8):
        agg = _edge_pass(u, packed, zeros)
        u = _combine_call(u, agg, agg, a, cf)

    out = _final_call(u, sqd, W.T, b.reshape(1, _D))
    return out[:_N]


# speedup vs baseline: 1.3260x; 1.3253x over previous
import functools

import jax
import jax.numpy as jnp
from jax import lax
from jax.experimental import pallas as pl
from jax.experimental.pallas import tpu as pltpu
from jax.experimental.pallas import tpu_sc as plsc

_N = 10000
_D = 128
_E = 320000
_NPAD = 10112
_NCORES = 2
_NSUB = 16
_NTILES = _NCORES * _NSUB
_KB = 2
_B = _KB * 128
_NB = 40
_EPT = _NB * _B
_RPT = _NPAD // _NSUB


@functools.partial(
    pl.kernel,
    mesh=plsc.VectorSubcoreMesh(core_axis_name="c", subcore_axis_name="s"),
    out_type=jax.ShapeDtypeStruct((_NCORES * _NPAD, _D), jnp.float32),
    scratch_types=[
        pltpu.VMEM((_NB, _B), jnp.int32),
        pltpu.VMEM((_B,), jnp.int32),
        pltpu.VMEM((_B,), jnp.int32),
        pltpu.VMEM((_B, _D), jnp.float32),
        pltpu.SemaphoreType.DMA,
        pltpu.VMEM_SHARED((_NPAD, _D), jnp.float32),
    ],
)
def _edge_pass(u_hbm, pk_hbm, z_hbm, agg_hbm, pk_v, cst, rst, buf, gsem, acc):
    c = lax.axis_index("c")
    s = lax.axis_index("s")
    w = c * _NSUB + s
    base = s * _RPT

    pltpu.sync_copy(z_hbm.at[pl.ds(base, _RPT)], acc.at[pl.ds(base, _RPT)])
    pltpu.sync_copy(pk_hbm.at[w], pk_v)

    plsc.subcore_barrier()

    def _body(g, carry):
        for k in range(_B // 16):
            pk = pk_v[g, pl.ds(k * 16, 16)]
            cst[pl.ds(k * 16, 16)] = pk & 0xFFFF
            rst[pl.ds(k * 16, 16)] = lax.shift_right_logical(pk, 16)
        pltpu.async_copy(u_hbm.at[cst], buf, gsem).wait()
        pltpu.sync_copy(buf, acc.at[rst], add=True)
        return carry

    lax.fori_loop(0, _NB, _body, 0)

    plsc.subcore_barrier()

    pltpu.sync_copy(acc.at[pl.ds(base, _RPT)],
                    agg_hbm.at[pl.ds(c * _NPAD + base, _RPT)])


_R = 2528
_GRID = _NPAD // _R


def _prep_body(feat_b, agg0_b, agg1_b, u0_b, a_b, c_b, sqd_b):
    deg = agg0_b[...] + agg1_b[...] + 1.0
    dinv = lax.rsqrt(deg)
    u0_b[...] = feat_b[...] * dinv
    inv = 1.0 / deg
    a_b[...] = 0.5 + 0.5 * inv
    c_b[...] = 0.5 * inv
    sqd_b[...] = deg * dinv


def _combine_body(u_b, agg0_b, agg1_b, a_b, c_b, out_b):
    out_b[...] = a_b[...] * u_b[...] + c_b[...] * (agg0_b[...] + agg1_b[...])


def _final_body(u_b, sqd_b, wt_b, bias_b, out_b):
    h = u_b[...] * sqd_b[...]
    out_b[...] = jnp.dot(h, wt_b[...],
                         preferred_element_type=jnp.float32) + bias_b[...]


def _row_spec():
    return pl.BlockSpec((_R, _D), lambda i: (i, 0))


def _agg_specs():
    return [pl.BlockSpec((_R, _D), lambda i: (i, 0)),
            pl.BlockSpec((_R, _D), lambda i: (i + _GRID, 0))]


_prep_call = pl.pallas_call(
    _prep_body,
    grid=(_GRID,),
    in_specs=[_row_spec()] + _agg_specs(),
    out_specs=[_row_spec()] * 4,
    out_shape=[jax.ShapeDtypeStruct((_NPAD, _D), jnp.float32)] * 4,
)

_combine_call = pl.pallas_call(
    _combine_body,
    grid=(_GRID,),
    in_specs=[_row_spec()] + _agg_specs() + [_row_spec(), _row_spec()],
    out_specs=_row_spec(),
    out_shape=jax.ShapeDtypeStruct((_NPAD, _D), jnp.float32),
)

_final_call = pl.pallas_call(
    _final_body,
    grid=(_GRID,),
    in_specs=[_row_spec(), _row_spec(),
              pl.BlockSpec((_D, _D), lambda i: (0, 0)),
              pl.BlockSpec((1, _D), lambda i: (0, 0))],
    out_specs=_row_spec(),
    out_shape=jax.ShapeDtypeStruct((_NPAD, _D), jnp.float32),
)


def kernel(feat, edge_index, W, b):
    row = edge_index[0]
    col = edge_index[1]
    pad = _NTILES * _EPT - _E
    sink = jnp.full((pad,), _NPAD - 1, jnp.int32)
    packed = jnp.concatenate(
        [(row << 16) | col, (sink << 16) | sink]).reshape(_NTILES, _NB, _B)

    featp = jnp.pad(feat, ((0, _NPAD - _N), (0, 0)))
    ones = jnp.pad(jnp.ones((_N, _D), jnp.float32), ((0, _NPAD - _N), (0, 0)))
    zeros = jnp.zeros((_NPAD, _D), jnp.float32)

    agg_deg = _edge_pass(ones, packed, zeros)
    u, a, cf, sqd = _prep_call(featp, agg_deg, agg_deg)

    for _ in range(8):
        agg = _edge_pass(u, packed, zeros)
        u = _combine_call(u, agg, agg, a, cf)

    out = _final_call(u, sqd, W.T, b.reshape(1, _D))
    return out[:_N]


# speedup vs baseline: 1.3268x; 1.0006x over previous
import functools

import jax
import jax.numpy as jnp
from jax import lax
from jax.experimental import pallas as pl
from jax.experimental.pallas import tpu as pltpu
from jax.experimental.pallas import tpu_sc as plsc

_N = 10000
_D = 128
_E = 320000
_NPAD = 10112
_NCORES = 2
_NSUB = 16
_NTILES = _NCORES * _NSUB
_B = 128
_NSLOT = 2
_NPAIR = 40
_NB = _NSLOT * _NPAIR
_EPT = _NB * _B
_RPT = _NPAD // _NSUB


@functools.partial(
    pl.kernel,
    mesh=plsc.VectorSubcoreMesh(core_axis_name="c", subcore_axis_name="s"),
    out_type=jax.ShapeDtypeStruct((_NCORES * _NPAD, _D), jnp.float32),
    scratch_types=[
        pltpu.VMEM((_NB, _B), jnp.int32),
        pltpu.VMEM((_NSLOT, _B), jnp.int32),
        pltpu.VMEM((_NSLOT, _B), jnp.int32),
        pltpu.VMEM((_NSLOT, _B, _D), jnp.float32),
        pltpu.SemaphoreType.DMA,
        pltpu.SemaphoreType.DMA,
        pltpu.VMEM_SHARED((_NPAD, _D), jnp.float32),
    ],
)
def _edge_pass(u_hbm, pk_hbm, z_hbm, agg_hbm,
               pk_v, cst, rst, bufs, gsem, ssem, acc):
    c = lax.axis_index("c")
    s = lax.axis_index("s")
    w = c * _NSUB + s
    base = s * _RPT

    pltpu.sync_copy(z_hbm.at[pl.ds(base, _RPT)], acc.at[pl.ds(base, _RPT)])
    pltpu.sync_copy(pk_hbm.at[w], pk_v)

    plsc.subcore_barrier()

    def _body(jp, carry):
        for b in range(_NSLOT):
            g = jp * _NSLOT + b
            for k in range(_B // 16):
                pk = pk_v[g, pl.ds(k * 16, 16)]
                cst[b, pl.ds(k * 16, 16)] = pk & 0xFFFF
                rst[b, pl.ds(k * 16, 16)] = lax.shift_right_logical(pk, 16)
        for b in range(_NSLOT):
            pltpu.async_copy(u_hbm.at[cst.at[b]], bufs.at[b], gsem)
        for b in range(_NSLOT):
            pltpu.make_async_copy(
                u_hbm.at[cst.at[b]], bufs.at[b], gsem).wait()
        for b in range(_NSLOT):
            pltpu.async_copy(bufs.at[b], acc.at[rst.at[b]], ssem, add=True)
        for b in range(_NSLOT):
            pltpu.make_async_copy(
                bufs.at[b], acc.at[rst.at[b]], ssem).wait()
        return carry

    lax.fori_loop(0, _NPAIR, _body, 0)

    plsc.subcore_barrier()

    pltpu.sync_copy(acc.at[pl.ds(base, _RPT)],
                    agg_hbm.at[pl.ds(c * _NPAD + base, _RPT)])


_R = 2528
_GRID = _NPAD // _R


def _prep_body(feat_b, agg0_b, agg1_b, u0_b, a_b, c_b, sqd_b):
    deg = agg0_b[...] + agg1_b[...] + 1.0
    dinv = lax.rsqrt(deg)
    u0_b[...] = feat_b[...] * dinv
    inv = 1.0 / deg
    a_b[...] = 0.5 + 0.5 * inv
    c_b[...] = 0.5 * inv
    sqd_b[...] = deg * dinv


def _combine_body(u_b, agg0_b, agg1_b, a_b, c_b, out_b):
    out_b[...] = a_b[...] * u_b[...] + c_b[...] * (agg0_b[...] + agg1_b[...])


def _final_body(u_b, sqd_b, wt_b, bias_b, out_b):
    h = u_b[...] * sqd_b[...]
    out_b[...] = jnp.dot(h, wt_b[...],
                         preferred_element_type=jnp.float32) + bias_b[...]


def _row_spec():
    return pl.BlockSpec((_R, _D), lambda i: (i, 0))


def _agg_specs():
    return [pl.BlockSpec((_R, _D), lambda i: (i, 0)),
            pl.BlockSpec((_R, _D), lambda i: (i + _GRID, 0))]


_prep_call = pl.pallas_call(
    _prep_body,
    grid=(_GRID,),
    in_specs=[_row_spec()] + _agg_specs(),
    out_specs=[_row_spec()] * 4,
    out_shape=[jax.ShapeDtypeStruct((_NPAD, _D), jnp.float32)] * 4,
)

_combine_call = pl.pallas_call(
    _combine_body,
    grid=(_GRID,),
    in_specs=[_row_spec()] + _agg_specs() + [_row_spec(), _row_spec()],
    out_specs=_row_spec(),
    out_shape=jax.ShapeDtypeStruct((_NPAD, _D), jnp.float32),
)

_final_call = pl.pallas_call(
    _final_body,
    grid=(_GRID,),
    in_specs=[_row_spec(), _row_spec(),
              pl.BlockSpec((_D, _D), lambda i: (0, 0)),
              pl.BlockSpec((1, _D), lambda i: (0, 0))],
    out_specs=_row_spec(),
    out_shape=jax.ShapeDtypeStruct((_NPAD, _D), jnp.float32),
)


def kernel(feat, edge_index, W, b):
    row = edge_index[0]
    col = edge_index[1]
    pad = _NTILES * _EPT - _E
    sink = jnp.full((pad,), _NPAD - 1, jnp.int32)
    packed = jnp.concatenate(
        [(row << 16) | col, (sink << 16) | sink]).reshape(_NTILES, _NB, _B)

    featp = jnp.pad(feat, ((0, _NPAD - _N), (0, 0)))
    ones = jnp.pad(jnp.ones((_N, _D), jnp.float32), ((0, _NPAD - _N), (0, 0)))
    zeros = jnp.zeros((_NPAD, _D), jnp.float32)

    agg_deg = _edge_pass(ones, packed, zeros)
    u, a, cf, sqd = _prep_call(featp, agg_deg, agg_deg)

    for _ in range(8):
        agg = _edge_pass(u, packed, zeros)
        u = _combine_call(u, agg, agg, a, cf)

    out = _final_call(u, sqd, W.T, b.reshape(1, _D))
    return out[:_N]


# speedup vs baseline: 1.4515x; 1.0940x over previous
import functools

import jax
import jax.numpy as jnp
from jax import lax
from jax.experimental import pallas as pl
from jax.experimental.pallas import tpu as pltpu
from jax.experimental.pallas import tpu_sc as plsc

_N = 10000
_D = 128
_E = 320000
_NPAD = 10112
_NCORES = 2
_NSUB = 16
_NTILES = _NCORES * _NSUB
_B = 128
_NSLOT = 2
_NPAIR = 40
_NB = _NSLOT * _NPAIR
_EPT = _NB * _B
_RPT = _NPAD // _NSUB


@functools.partial(
    pl.kernel,
    mesh=plsc.VectorSubcoreMesh(core_axis_name="c", subcore_axis_name="s"),
    out_type=jax.ShapeDtypeStruct((_NCORES * _NPAD, _D), jnp.float32),
    scratch_types=[
        pltpu.VMEM((_NB // 2, _B), jnp.int32),
        pltpu.VMEM((_NB // 2, _B), jnp.int32),
        pltpu.VMEM((_NSLOT, _B, _D), jnp.float32),
        pltpu.SemaphoreType.DMA,
        pltpu.SemaphoreType.DMA,
        pltpu.VMEM_SHARED((_NPAD, _D), jnp.float32),
    ],
)
def _edge_pass(u_hbm, col_hbm, row_hbm, z_hbm, agg_hbm,
               col_v, row_v, bufs, gsem, ssem, acc):
    c = lax.axis_index("c")
    s = lax.axis_index("s")
    w = c * _NSUB + s
    base = s * _RPT

    pltpu.sync_copy(z_hbm.at[pl.ds(base, _RPT)], acc.at[pl.ds(base, _RPT)])

    plsc.subcore_barrier()

    for half in range(2):
        pltpu.sync_copy(col_hbm.at[2 * w + half], col_v)
        pltpu.sync_copy(row_hbm.at[2 * w + half], row_v)

        def _body(jp, carry):
            for b in range(_NSLOT):
                g = jp * _NSLOT + b
                pltpu.async_copy(u_hbm.at[col_v.at[g]], bufs.at[b], gsem)
            for b in range(_NSLOT):
                g = jp * _NSLOT + b
                pltpu.make_async_copy(
                    u_hbm.at[col_v.at[g]], bufs.at[b], gsem).wait()
            for b in range(_NSLOT):
                g = jp * _NSLOT + b
                pltpu.async_copy(
                    bufs.at[b], acc.at[row_v.at[g]], ssem, add=True)
            for b in range(_NSLOT):
                g = jp * _NSLOT + b
                pltpu.make_async_copy(
                    bufs.at[b], acc.at[row_v.at[g]], ssem).wait()
            return carry

        lax.fori_loop(0, _NPAIR // 2, _body, 0)

    plsc.subcore_barrier()

    pltpu.sync_copy(acc.at[pl.ds(base, _RPT)],
                    agg_hbm.at[pl.ds(c * _NPAD + base, _RPT)])


_R = 2528
_GRID = _NPAD // _R


def _prep_body(feat_b, agg0_b, agg1_b, u0_b, a_b, c_b, sqd_b):
    deg = agg0_b[...] + agg1_b[...] + 1.0
    dinv = lax.rsqrt(deg)
    u0_b[...] = feat_b[...] * dinv
    inv = 1.0 / deg
    a_b[...] = 0.5 + 0.5 * inv
    c_b[...] = 0.5 * inv
    sqd_b[...] = deg * dinv


def _combine_body(u_b, agg0_b, agg1_b, a_b, c_b, out_b):
    out_b[...] = a_b[...] * u_b[...] + c_b[...] * (agg0_b[...] + agg1_b[...])


def _final_body(u_b, sqd_b, wt_b, bias_b, out_b):
    h = u_b[...] * sqd_b[...]
    out_b[...] = jnp.dot(h, wt_b[...],
                         preferred_element_type=jnp.float32) + bias_b[...]


def _row_spec():
    return pl.BlockSpec((_R, _D), lambda i: (i, 0))


def _agg_specs():
    return [pl.BlockSpec((_R, _D), lambda i: (i, 0)),
            pl.BlockSpec((_R, _D), lambda i: (i + _GRID, 0))]


_prep_call = pl.pallas_call(
    _prep_body,
    grid=(_GRID,),
    in_specs=[_row_spec()] + _agg_specs(),
    out_specs=[_row_spec()] * 4,
    out_shape=[jax.ShapeDtypeStruct((_NPAD, _D), jnp.float32)] * 4,
)

_combine_call = pl.pallas_call(
    _combine_body,
    grid=(_GRID,),
    in_specs=[_row_spec()] + _agg_specs() + [_row_spec(), _row_spec()],
    out_specs=_row_spec(),
    out_shape=jax.ShapeDtypeStruct((_NPAD, _D), jnp.float32),
)

_final_call = pl.pallas_call(
    _final_body,
    grid=(_GRID,),
    in_specs=[_row_spec(), _row_spec(),
              pl.BlockSpec((_D, _D), lambda i: (0, 0)),
              pl.BlockSpec((1, _D), lambda i: (0, 0))],
    out_specs=_row_spec(),
    out_shape=jax.ShapeDtypeStruct((_NPAD, _D), jnp.float32),
)


def kernel(feat, edge_index, W, b):
    row = edge_index[0]
    col = edge_index[1]
    pad = _NTILES * _EPT - _E
    sink = jnp.full((pad,), _NPAD - 1, jnp.int32)
    colp = jnp.concatenate([col, sink]).reshape(_NTILES * 2, _NB // 2, _B)
    rowp = jnp.concatenate([row, sink]).reshape(_NTILES * 2, _NB // 2, _B)

    featp = jnp.pad(feat, ((0, _NPAD - _N), (0, 0)))
    ones = jnp.pad(jnp.ones((_N, _D), jnp.float32), ((0, _NPAD - _N), (0, 0)))
    zeros = jnp.zeros((_NPAD, _D), jnp.float32)

    agg_deg = _edge_pass(ones, colp, rowp, zeros)
    u, a, cf, sqd = _prep_call(featp, agg_deg, agg_deg)

    for _ in range(8):
        agg = _edge_pass(u, colp, rowp, zeros)
        u = _combine_call(u, agg, agg, a, cf)

    out = _final_call(u, sqd, W.T, b.reshape(1, _D))
    return out[:_N]


# speedup vs baseline: 1.5276x; 1.0524x over previous
import functools

import jax
import jax.numpy as jnp
from jax import lax
from jax.experimental import pallas as pl
from jax.experimental.pallas import tpu as pltpu
from jax.experimental.pallas import tpu_sc as plsc

_N = 10000
_D = 128
_E = 320000
_NPAD = 10112
_NCORES = 2
_NSUB = 16
_NTILES = _NCORES * _NSUB
_B = 128
_NSLOT = 2
_NPAIR = 40
_NB = _NSLOT * _NPAIR
_EPT = _NB * _B
_RPT = _NPAD // _NSUB


@functools.partial(
    pl.kernel,
    mesh=plsc.VectorSubcoreMesh(core_axis_name="c", subcore_axis_name="s"),
    out_type=jax.ShapeDtypeStruct((_NCORES * _NPAD, _D), jnp.float32),
    scratch_types=[
        pltpu.VMEM((_NB // 4, _B), jnp.int32),
        pltpu.VMEM((_NB // 4, _B), jnp.int32),
        pltpu.VMEM((_NSLOT, _B, _D), jnp.float32),
        pltpu.SemaphoreType.DMA,
        pltpu.SemaphoreType.DMA,
        pltpu.VMEM_SHARED((_NPAD, _D), jnp.float32),
    ],
)
def _edge_pass(u_hbm, col_hbm, row_hbm, z_hbm, agg_hbm,
               col_v, row_v, bufs, gsem, ssem, acc):
    c = lax.axis_index("c")
    s = lax.axis_index("s")
    w = c * _NSUB + s
    base = s * _RPT

    pltpu.sync_copy(z_hbm.at[pl.ds(base, _RPT)], acc.at[pl.ds(base, _RPT)])

    plsc.subcore_barrier()

    _NQ = _NB // 4

    def _gather(g, b):
        return pltpu.make_async_copy(
            u_hbm.at[col_v.at[g]], bufs.at[b], gsem)

    def _scatter(g, b):
        return pltpu.make_async_copy(
            bufs.at[b], acc.at[row_v.at[g]], ssem)

    for q in range(4):
        pltpu.sync_copy(col_hbm.at[4 * w + q], col_v)
        pltpu.sync_copy(row_hbm.at[4 * w + q], row_v)

        _gather(0, 0).start()

        def _body(jp, carry):
            for b in range(2):
                g = 2 * jp + b
                _gather(g, b).wait()
                _scatter(g, b).start(add=True)

                @pl.when(g > 0)
                def _():
                    _scatter(g - 1, 1 - b).wait()

                @pl.when(g < _NQ - 1)
                def _():
                    _gather(g + 1, 1 - b).start()
            return carry

        lax.fori_loop(0, _NQ // 2, _body, 0)
        _scatter(_NQ - 1, 1).wait()

    plsc.subcore_barrier()

    pltpu.sync_copy(acc.at[pl.ds(base, _RPT)],
                    agg_hbm.at[pl.ds(c * _NPAD + base, _RPT)])


_R = 2528
_GRID = _NPAD // _R


def _prep_body(feat_b, agg0_b, agg1_b, u0_b, a_b, c_b, sqd_b):
    deg = agg0_b[...] + agg1_b[...] + 1.0
    dinv = lax.rsqrt(deg)
    u0_b[...] = feat_b[...] * dinv
    inv = 1.0 / deg
    a_b[...] = 0.5 + 0.5 * inv
    c_b[...] = 0.5 * inv
    sqd_b[...] = deg * dinv


def _combine_body(u_b, agg0_b, agg1_b, a_b, c_b, out_b):
    out_b[...] = a_b[...] * u_b[...] + c_b[...] * (agg0_b[...] + agg1_b[...])


def _final_body(u_b, sqd_b, wt_b, bias_b, out_b):
    h = u_b[...] * sqd_b[...]
    out_b[...] = jnp.dot(h, wt_b[...],
                         preferred_element_type=jnp.float32) + bias_b[...]


def _row_spec():
    return pl.BlockSpec((_R, _D), lambda i: (i, 0))


def _agg_specs():
    return [pl.BlockSpec((_R, _D), lambda i: (i, 0)),
            pl.BlockSpec((_R, _D), lambda i: (i + _GRID, 0))]


_prep_call = pl.pallas_call(
    _prep_body,
    grid=(_GRID,),
    in_specs=[_row_spec()] + _agg_specs(),
    out_specs=[_row_spec()] * 4,
    out_shape=[jax.ShapeDtypeStruct((_NPAD, _D), jnp.float32)] * 4,
)

_combine_call = pl.pallas_call(
    _combine_body,
    grid=(_GRID,),
    in_specs=[_row_spec()] + _agg_specs() + [_row_spec(), _row_spec()],
    out_specs=_row_spec(),
    out_shape=jax.ShapeDtypeStruct((_NPAD, _D), jnp.float32),
)

_final_call = pl.pallas_call(
    _final_body,
    grid=(_GRID,),
    in_specs=[_row_spec(), _row_spec(),
              pl.BlockSpec((_D, _D), lambda i: (0, 0)),
              pl.BlockSpec((1, _D), lambda i: (0, 0))],
    out_specs=_row_spec(),
    out_shape=jax.ShapeDtypeStruct((_NPAD, _D), jnp.float32),
)


def kernel(feat, edge_index, W, b):
    row = edge_index[0]
    col = edge_index[1]
    pad = _NTILES * _EPT - _E
    sink = jnp.full((pad,), _NPAD - 1, jnp.int32)
    colp = jnp.concatenate([col, sink]).reshape(_NTILES * 4, _NB // 4, _B)
    rowp = jnp.concatenate([row, sink]).reshape(_NTILES * 4, _NB // 4, _B)

    featp = jnp.pad(feat, ((0, _NPAD - _N), (0, 0)))
    ones = jnp.pad(jnp.ones((_N, _D), jnp.float32), ((0, _NPAD - _N), (0, 0)))
    zeros = jnp.zeros((_NPAD, _D), jnp.float32)

    agg_deg = _edge_pass(ones, colp, rowp, zeros)
    u, a, cf, sqd = _prep_call(featp, agg_deg, agg_deg)

    for _ in range(8):
        agg = _edge_pass(u, colp, rowp, zeros)
        u = _combine_call(u, agg, agg, a, cf)

    out = _final_call(u, sqd, W.T, b.reshape(1, _D))
    return out[:_N]


# speedup vs baseline: 2.1081x; 1.3800x over previous
import functools

import jax
import jax.numpy as jnp
from jax import lax
from jax.experimental import pallas as pl
from jax.experimental.pallas import tpu as pltpu
from jax.experimental.pallas import tpu_sc as plsc

_N = 10000
_D = 128
_E = 320000
_NPAD = 10240
_NCORES = 2
_NSUB = 16
_NTILES = _NCORES * _NSUB
_B = 128
_NB = 79
_EPT = _NB * _B
_RPT = _NPAD // _NSUB


@functools.partial(
    pl.kernel,
    mesh=plsc.VectorSubcoreMesh(core_axis_name="c", subcore_axis_name="s"),
    out_type=jax.ShapeDtypeStruct((_NCORES * _NPAD, _D), jnp.float32),
    scratch_types=[
        pltpu.VMEM((_NB, _B), jnp.int32),
        pltpu.VMEM((_NB, _B), jnp.int32),
        pltpu.VMEM((_B, _D), jnp.float32),
        pltpu.SemaphoreType.DMA,
        pltpu.VMEM_SHARED((_NPAD, _D), jnp.float32),
    ],
)
def _edge_pass(u_hbm, col_hbm, row_hbm, z_hbm, agg_hbm,
               col_v, row_v, buf, gsem, acc):
    c = lax.axis_index("c")
    s = lax.axis_index("s")
    w = c * _NSUB + s
    base = s * _RPT

    pltpu.sync_copy(z_hbm.at[pl.ds(base, _RPT)], acc.at[pl.ds(base, _RPT)])

    plsc.subcore_barrier()

    pltpu.sync_copy(col_hbm.at[w], col_v)
    pltpu.sync_copy(row_hbm.at[w], row_v)

    def _body(g, carry):
        pltpu.async_copy(u_hbm.at[col_v.at[g]], buf, gsem).wait()
        pltpu.sync_copy(buf, acc.at[row_v.at[g]], add=True)
        return carry

    lax.fori_loop(0, _NB, _body, 0)

    plsc.subcore_barrier()

    pltpu.sync_copy(acc.at[pl.ds(base, _RPT)],
                    agg_hbm.at[pl.ds(c * _NPAD + base, _RPT)])


_R = 2560
_GRID = _NPAD // _R


def _prep_body(feat_b, agg0_b, agg1_b, u0_b, a_b, c_b, sqd_b):
    deg = agg0_b[...] + agg1_b[...] + 1.0
    dinv = lax.rsqrt(deg)
    u0_b[...] = feat_b[...] * dinv
    inv = 1.0 / deg
    a_b[...] = 0.5 + 0.5 * inv
    c_b[...] = 0.5 * inv
    sqd_b[...] = deg * dinv


def _combine_body(u_b, agg0_b, agg1_b, a_b, c_b, out_b):
    out_b[...] = a_b[...] * u_b[...] + c_b[...] * (agg0_b[...] + agg1_b[...])


def _final_body(u_b, sqd_b, wt_b, bias_b, out_b):
    h = u_b[...] * sqd_b[...]
    out_b[...] = jnp.dot(h, wt_b[...],
                         preferred_element_type=jnp.float32) + bias_b[...]


def _row_spec():
    return pl.BlockSpec((_R, _D), lambda i: (i, 0))


def _agg_specs():
    return [pl.BlockSpec((_R, _D), lambda i: (i, 0)),
            pl.BlockSpec((_R, _D), lambda i: (i + _GRID, 0))]


_prep_call = pl.pallas_call(
    _prep_body,
    grid=(_GRID,),
    in_specs=[_row_spec()] + _agg_specs(),
    out_specs=[_row_spec()] * 4,
    out_shape=[jax.ShapeDtypeStruct((_NPAD, _D), jnp.float32)] * 4,
)

_combine_call = pl.pallas_call(
    _combine_body,
    grid=(_GRID,),
    in_specs=[_row_spec()] + _agg_specs() + [_row_spec(), _row_spec()],
    out_specs=_row_spec(),
    out_shape=jax.ShapeDtypeStruct((_NPAD, _D), jnp.float32),
)

_final_call = pl.pallas_call(
    _final_body,
    grid=(_GRID,),
    in_specs=[_row_spec(), _row_spec(),
              pl.BlockSpec((_D, _D), lambda i: (0, 0)),
              pl.BlockSpec((1, _D), lambda i: (0, 0))],
    out_specs=_row_spec(),
    out_shape=jax.ShapeDtypeStruct((_NPAD, _D), jnp.float32),
)


def kernel(feat, edge_index, W, b):
    row = edge_index[0]
    col = edge_index[1]
    pad = _NTILES * _EPT - _E
    sink = jnp.full((pad,), _NPAD - 1, jnp.int32)
    colp = jnp.concatenate([col, sink]).reshape(_NTILES, _NB, _B)
    rowp = jnp.concatenate([row, sink]).reshape(_NTILES, _NB, _B)

    featp = jnp.pad(feat, ((0, _NPAD - _N), (0, 0)))
    ones = jnp.pad(jnp.ones((_N, _D), jnp.float32), ((0, _NPAD - _N), (0, 0)))
    zeros = jnp.zeros((_NPAD, _D), jnp.float32)

    agg_deg = _edge_pass(ones, colp, rowp, zeros)
    u, a, cf, sqd = _prep_call(featp, agg_deg, agg_deg)

    for _ in range(8):
        agg = _edge_pass(u, colp, rowp, zeros)
        u = _combine_call(u, agg, agg, a, cf)

    out = _final_call(u, sqd, W.T, b.reshape(1, _D))
    return out[:_N]
